# baseline (speedup 1.0000x reference)
import jax
import jax.numpy as jnp
from jax import lax
from jax.experimental import pallas as pl
from jax.experimental.pallas import tpu as pltpu

NUM_CLASSES = 80


def _body(tgt8_ref, tgt32_ref, x_ref, lp_ref, lt_ref, out_ref):
    x = x_ref[0]
    tgt8 = tgt8_ref[0]
    tgt32 = tgt32_ref[0]
    lp = lp_ref[0]
    lt = lt_ref[0]

    l640 = lax.broadcasted_iota(jnp.int32, (8, 640), 1)
    k8 = lax.broadcasted_iota(jnp.int32, (8, 640), 0)
    e80 = (l640 // 80 == k8).astype(jnp.float32)
    tgt640 = jnp.dot(tgt8.astype(jnp.float32), e80,
                     preferred_element_type=jnp.float32)
    clspat = (lax.broadcasted_iota(jnp.int32, (1, 640), 1) % 80 + 1)
    t = tgt640 == clspat.astype(jnp.float32)

    s2 = jnp.where(t, -2.0, 2.0)
    z = x * s2 - 1.0
    sp = jnp.maximum(z, 0.0) + jnp.log1p(jnp.exp(-jnp.abs(z)))
    w2 = jnp.where(t, 0.125, 0.375)
    cls_part = jnp.sum(w2 * sp)

    np_part = jnp.sum(jnp.where(tgt8 > 0, 1.0, 0.0))

    l128 = lax.broadcasted_iota(jnp.int32, (32, 128), 1)
    k32 = lax.broadcasted_iota(jnp.int32, (32, 128), 0)
    e4 = (l128 // 4 == k32).astype(jnp.float32)
    pos128 = jnp.dot(tgt32.astype(jnp.float32), e4,
                     preferred_element_type=jnp.float32) > 0.0

    d = lp - lt
    ad = jnp.abs(d)
    elem = jnp.where(ad < 1.0, 0.5 * d * d, ad - 0.5)
    loc_part = jnp.sum(jnp.where(pos128, elem, 0.0))

    @pl.when((pl.program_id(0) == 0) & (pl.program_id(1) == 0))
    def _():
        out_ref[0] = 0.0
        out_ref[1] = 0.0
        out_ref[2] = 0.0

    out_ref[0] += cls_part
    out_ref[1] += loc_part
    out_ref[2] += np_part


def kernel(loc_preds, loc_targets, cls_preds, cls_targets):
    b, a, _ = loc_preds.shape

    x3 = cls_preds.reshape(b, a // 8, 8 * NUM_CLASSES)
    lp3 = loc_preds.reshape(b, a // 32, 128)
    lt3 = loc_targets.reshape(b, a // 32, 128)
    t8 = cls_targets.reshape(b, a // 8, 8)
    t32 = cls_targets.reshape(b, a // 32, 32)

    ablk = 8192
    r8, r32 = ablk // 8, ablk // 32
    grid = (b, a // ablk)

    sums = pl.pallas_call(
        _body,
        grid=grid,
        in_specs=[
            pl.BlockSpec((1, r8, 8), lambda i, j: (i, j, 0)),
            pl.BlockSpec((1, r32, 32), lambda i, j: (i, j, 0)),
            pl.BlockSpec((1, r8, 8 * NUM_CLASSES), lambda i, j: (i, j, 0)),
            pl.BlockSpec((1, r32, 128), lambda i, j: (i, j, 0)),
            pl.BlockSpec((1, r32, 128), lambda i, j: (i, j, 0)),
        ],
        out_specs=pl.BlockSpec(memory_space=pltpu.SMEM),
        out_shape=jax.ShapeDtypeStruct((3,), jnp.float32),
    )(t8, t32, x3, lp3, lt3)

    return (sums[0] + sums[1]) / sums[2]


# speedup vs baseline: 5.2255x; 5.2255x over previous
import jax
import jax.numpy as jnp
from jax import lax
from jax.experimental import pallas as pl
from jax.experimental.pallas import tpu as pltpu

NUM_CLASSES = 80
_LN2 = 0.6931471805599453


def _body(tgt_ref, x_ref, lp_ref, lt_ref, out_ref):
    x = x_ref[0]
    tgt = tgt_ref[0]
    lp = lp_ref[0]
    lt = lt_ref[0]

    cls_id = lax.broadcasted_iota(jnp.int32, (NUM_CLASSES, 1), 0) + 1
    t = tgt == cls_id

    s1 = jnp.where(t, 1.0, -1.0)
    u = x * s1 + 0.5
    q = jnp.maximum(1.0 + jnp.tanh(u), 1e-38)
    lg = jnp.log(q)
    wc = jnp.where(t, 0.125, 0.375)
    cls_neg_part = jnp.sum(wc * lg)

    pos = tgt > 0
    np_part = jnp.sum(jnp.where(pos, 1.0, 0.0))

    d = lp - lt
    ad = jnp.abs(d)
    elem = jnp.where(ad < 1.0, 0.5 * d * d, ad - 0.5)
    loc_part = jnp.sum(jnp.where(pos, elem, 0.0))

    @pl.when((pl.program_id(0) == 0) & (pl.program_id(1) == 0))
    def _():
        out_ref[0] = 0.0
        out_ref[1] = 0.0
        out_ref[2] = 0.0

    out_ref[0] += cls_neg_part
    out_ref[1] += loc_part
    out_ref[2] += np_part


def kernel(loc_preds, loc_targets, cls_preds, cls_targets):
    b, a, _ = loc_preds.shape

    xt = cls_preds.transpose(0, 2, 1)
    lpt = loc_preds.transpose(0, 2, 1)
    ltt = loc_targets.transpose(0, 2, 1)
    tgt3 = cls_targets.reshape(b, 1, a)

    lblk = 8192
    grid = (b, a // lblk)

    sums = pl.pallas_call(
        _body,
        grid=grid,
        in_specs=[
            pl.BlockSpec((1, 1, lblk), lambda i, j: (i, 0, j)),
            pl.BlockSpec((1, NUM_CLASSES, lblk), lambda i, j: (i, 0, j)),
            pl.BlockSpec((1, 4, lblk), lambda i, j: (i, 0, j)),
            pl.BlockSpec((1, 4, lblk), lambda i, j: (i, 0, j)),
        ],
        out_specs=pl.BlockSpec(memory_space=pltpu.SMEM),
        out_shape=jax.ShapeDtypeStruct((3,), jnp.float32),
    )(tgt3, xt, lpt, ltt)

    num_pos = sums[2]
    n_el = b * a * NUM_CLASSES
    cls_loss = _LN2 * (0.375 * n_el - 0.25 * num_pos) - sums[0]
    return (cls_loss + sums[1]) / num_pos
